# baseline (speedup 1.0000x reference)
import functools

import jax
import jax.numpy as jnp
from jax import lax
from jax.experimental import pallas as pl
from jax.experimental.pallas import tpu as pltpu
from jax.experimental.pallas import tpu_sc as plsc

N = 10000
E = 320000
D = 132
DP = 144
EH = 32

NC = 2
NS = 16
NW = NC * NS
CHUNK = 128
EPW = 10240
NCHUNK = EPW // CHUNK
IG = 2
NIG = NCHUNK // IG
TGRP = NCHUNK // 8
PW = 80
E_PAD = NW * EPW
ROWS_PER_TILE = 632
N_STRIPED = NS * ROWS_PER_TILE
JUNK_ROW = N_STRIPED
AGG_ROWS = N_STRIPED + 8


def _edge_net_kernel(e_ref, We1_ref, be1_ref, We2_ref, be2_ref, out_ref):
    eh = jnp.maximum(
        jnp.dot(e_ref[...], We1_ref[...], preferred_element_type=jnp.float32)
        + be1_ref[...], 0.0)
    out_ref[...] = (
        jnp.dot(eh, We2_ref[...], preferred_element_type=jnp.float32)
        + be2_ref[...])


def _edge_net(e0, We1, be1, We2, be2):
    out = pl.pallas_call(
        _edge_net_kernel,
        out_shape=jax.ShapeDtypeStruct((1, D * D), jnp.float32),
    )(e0, We1, be1.reshape(1, EH), We2, be2.reshape(1, D * D))
    return out.reshape(D, D)


def _node_proj_kernel(x_ref, Wp_ref, bp_ref, At_ref, h_ref, pk_ref):
    h = jnp.maximum(
        jnp.dot(x_ref[...], Wp_ref[...], preferred_element_type=jnp.float32)
        + bp_ref[...], 0.0)
    h_ref[...] = h
    proj = jnp.dot(h, At_ref[...], preferred_element_type=jnp.float32)
    pp = jnp.concatenate(
        [proj, jnp.zeros((proj.shape[0], 2 * PW - D), jnp.float32)], axis=1)
    bits = jax.lax.bitcast_convert_type(pp, jnp.uint32) + jnp.uint32(0x8000)
    hi = bits[:, :PW] & jnp.uint32(0xFFFF0000)
    lo = bits[:, PW:] >> jnp.uint32(16)
    pk_ref[...] = jax.lax.bitcast_convert_type(hi | lo, jnp.int32)


def _node_proj(x, W_proj, b_proj, A_t):
    blk = 2000
    grid = (N // blk,)
    return pl.pallas_call(
        _node_proj_kernel,
        grid=grid,
        in_specs=[
            pl.BlockSpec((blk, D), lambda i: (i, 0)),
            pl.BlockSpec((D, D), lambda i: (0, 0)),
            pl.BlockSpec((1, D), lambda i: (0, 0)),
            pl.BlockSpec((D, D), lambda i: (0, 0)),
        ],
        out_specs=[
            pl.BlockSpec((blk, D), lambda i: (i, 0)),
            pl.BlockSpec((blk, PW), lambda i: (i, 0)),
        ],
        out_shape=[
            jax.ShapeDtypeStruct((N, D), jnp.float32),
            jax.ShapeDtypeStruct((N, PW), jnp.int32),
        ],
    )(x, W_proj, b_proj.reshape(1, D), A_t)


NBUF = 2


def _sc_body(proj_hbm, src_hbm, dst_hbm, zeros_hbm, out_hbm,
             sg0, sg1, dg0, dg1, r0, r1, f32b, agg_sh,
             is0, is1, id0, id1, g0, g1):
    c = lax.axis_index("c")
    s = lax.axis_index("s")
    w = c * NS + s
    rows = [r0, r1]
    sgb = [sg0, sg1]
    dgb = [dg0, dg1]
    isem = [is0, is1]
    jsem = [id0, id1]
    gsem = [g0, g1]

    pltpu.sync_copy(zeros_hbm.at[pl.ds(s * ROWS_PER_TILE, ROWS_PER_TILE)],
                    agg_sh.at[pl.ds(s * ROWS_PER_TILE, ROWS_PER_TILE)])
    @pl.when(s == 0)
    def _():
        pltpu.sync_copy(zeros_hbm.at[pl.ds(0, AGG_ROWS - N_STRIPED)],
                        agg_sh.at[pl.ds(N_STRIPED, AGG_ROWS - N_STRIPED)])

    def idx_off(g, j):
        return w * CHUNK + (g * IG + j) * NW * CHUNK

    def idx_fetch(g, q):
        for j in range(IG):
            pltpu.async_copy(src_hbm.at[pl.ds(idx_off(g, j), CHUNK)],
                             sgb[q].at[j], isem[q])
            pltpu.async_copy(dst_hbm.at[pl.ds(idx_off(g, j), CHUNK)],
                             dgb[q].at[j], jsem[q])

    def idx_wait(g, q):
        for j in range(IG):
            pltpu.make_async_copy(src_hbm.at[pl.ds(idx_off(g, j), CHUNK)],
                                  sgb[q].at[j], isem[q]).wait()
            pltpu.make_async_copy(dst_hbm.at[pl.ds(idx_off(g, j), CHUNK)],
                                  dgb[q].at[j], jsem[q]).wait()

    def gather_start(t, k):
        q = (k // 2) % 2
        r = k % 2
        g = 4 * t + k // 2
        if r == 0:
            idx_wait(g, q)
        if r == 1:
            @pl.when(g + 1 < NIG)
            def _():
                idx_fetch(g + 1, 1 - q)
        pltpu.async_copy(proj_hbm.at[sgb[q].at[r]], rows[k % 2], gsem[k % 2])

    def gather_wait(k):
        q = (k // 2) % 2
        pltpu.make_async_copy(proj_hbm.at[sgb[q].at[k % 2]], rows[k % 2],
                              gsem[k % 2]).wait()

    def unpack_rows(k):
        bfb = rows[k % 2]

        def row(i, carry):
            for v in range(PW // 16):
                wv = bfb[i, pl.ds(16 * v, 16)]
                f32b[i, pl.ds(16 * v, 16)] = jax.lax.bitcast_convert_type(
                    wv & jnp.int32(-65536), jnp.float32)
                if v < (DP - PW) // 16:
                    f32b[i, pl.ds(PW + 16 * v, 16)] = (
                        jax.lax.bitcast_convert_type(wv << 16, jnp.float32))
            return carry

        lax.fori_loop(0, CHUNK, row, 0)

    idx_fetch(0, 0)
    plsc.subcore_barrier()
    gather_start(0, 0)
    gather_start(0, 1)

    def octet(t, carry):
        for k in range(8):
            gather_wait(k)
            unpack_rows(k)
            pltpu.sync_copy(f32b,
                            agg_sh.at[dgb[(k // 2) % 2].at[k % 2]], add=True)
            if k < 6:
                gather_start(t, k + 2)
            else:
                @pl.when(t < TGRP - 1)
                def _():
                    gather_start(t + 1, k - 6)
        return carry

    lax.fori_loop(0, TGRP, octet, 0)
    plsc.subcore_barrier()

    pltpu.sync_copy(agg_sh.at[pl.ds(s * ROWS_PER_TILE, ROWS_PER_TILE)],
                    out_hbm.at[c].at[pl.ds(s * ROWS_PER_TILE, ROWS_PER_TILE)])


def _sc_aggregate(proj_pad, src_pad, dst_pad, zeros_hbm):
    mesh = plsc.VectorSubcoreMesh(core_axis_name="c", subcore_axis_name="s")
    k = pl.kernel(
        _sc_body,
        out_type=jax.ShapeDtypeStruct((NC, N_STRIPED, DP), jnp.float32),
        mesh=mesh,
        scratch_types=(
            [pltpu.VMEM((IG, CHUNK), jnp.int32)] * 4
            + [pltpu.VMEM((CHUNK, PW), jnp.int32)] * NBUF
            + [pltpu.VMEM((CHUNK, DP), jnp.float32)]
            + [pltpu.VMEM_SHARED((AGG_ROWS, DP), jnp.float32)]
            + [pltpu.SemaphoreType.DMA] * 6
        ),
        compiler_params=pltpu.CompilerParams(use_tc_tiling_on_sc=False),
    )
    return k(proj_pad, src_pad, dst_pad, zeros_hbm)


def _gru_kernel(p_ref, h_ref,
                Wir_ref, Wiz_ref, Win_ref, bi_ref,
                Whr_ref, Whz_ref, Whn_ref, bh_ref, out_ref):
    m = jnp.maximum(p_ref[0, :, :D] + p_ref[1, :, :D], 0.0)
    h = h_ref[...]
    dot = lambda a, b: jnp.dot(a, b, preferred_element_type=jnp.float32)
    r = jax.nn.sigmoid(dot(m, Wir_ref[...]) + bi_ref[0:1, :]
                       + dot(h, Whr_ref[...]) + bh_ref[0:1, :])
    z = jax.nn.sigmoid(dot(m, Wiz_ref[...]) + bi_ref[1:2, :]
                       + dot(h, Whz_ref[...]) + bh_ref[1:2, :])
    n = jnp.tanh(dot(m, Win_ref[...]) + bi_ref[2:3, :]
                 + r * (dot(h, Whn_ref[...]) + bh_ref[2:3, :]))
    out_ref[...] = (1.0 - z) * n + z * h


def _gru(partials, h, W_ih, b_ih, W_hh, b_hh):
    blk = 2000
    grid = (N // blk,)
    Wir, Wiz, Win = W_ih[:, :D], W_ih[:, D:2 * D], W_ih[:, 2 * D:]
    Whr, Whz, Whn = W_hh[:, :D], W_hh[:, D:2 * D], W_hh[:, 2 * D:]
    bi = b_ih.reshape(3, D)
    bh = b_hh.reshape(3, D)
    full = lambda shape: pl.BlockSpec(shape, lambda i: tuple(0 for _ in shape))
    return pl.pallas_call(
        _gru_kernel,
        grid=grid,
        in_specs=[
            pl.BlockSpec((2, blk, DP), lambda i: (0, i, 0)),
            pl.BlockSpec((blk, D), lambda i: (i, 0)),
            full((D, D)), full((D, D)), full((D, D)), full((3, D)),
            full((D, D)), full((D, D)), full((D, D)), full((3, D)),
        ],
        out_specs=pl.BlockSpec((blk, D), lambda i: (i, 0)),
        out_shape=jax.ShapeDtypeStruct((N, D), jnp.float32),
    )(partials, h, Wir, Wiz, Win, bi, Whr, Whz, Whn, bh)


@jax.jit
def kernel(x, edge_index, edge_attr, W_proj, b_proj, We1, be1, We2, be2,
           W_ih, b_ih, W_hh, b_hh):
    A = _edge_net(edge_attr[:1], We1, be1, We2, be2)
    h, proj_pad = _node_proj(x, W_proj, b_proj, A.T)

    pad = E_PAD - E
    src_pad = jnp.concatenate([edge_index[0], jnp.zeros((pad,), jnp.int32)])
    dst_pad = jnp.concatenate(
        [edge_index[1], jnp.full((pad,), JUNK_ROW, jnp.int32)])
    zeros_hbm = jnp.zeros((N_STRIPED, DP), jnp.float32)

    partials = _sc_aggregate(proj_pad, src_pad, dst_pad, zeros_hbm)
    return _gru(partials, h, W_ih, b_ih, W_hh, b_hh)


# speedup vs baseline: 1.1458x; 1.1458x over previous
import functools

import jax
import jax.numpy as jnp
from jax import lax
from jax.experimental import pallas as pl
from jax.experimental.pallas import tpu as pltpu
from jax.experimental.pallas import tpu_sc as plsc

N = 10000
E = 320000
D = 132
DP = 144
EH = 32

NC = 2
NS = 16
NW = NC * NS
CHUNK = 128
EPW = 10240
NCHUNK = EPW // CHUNK
IG = 4
NIG = NCHUNK // IG
TGRP = NCHUNK // 8
E_PAD = NW * EPW
ROWS_PER_TILE = 632
N_STRIPED = NS * ROWS_PER_TILE
JUNK_ROW = N_STRIPED
AGG_ROWS = N_STRIPED + 8


def _edge_net_kernel(e_ref, We1_ref, be1_ref, We2_ref, be2_ref, out_ref):
    eh = jnp.maximum(
        jnp.dot(e_ref[...], We1_ref[...], preferred_element_type=jnp.float32)
        + be1_ref[...], 0.0)
    out_ref[...] = (
        jnp.dot(eh, We2_ref[...], preferred_element_type=jnp.float32)
        + be2_ref[...])


def _edge_net(e0, We1, be1, We2, be2):
    out = pl.pallas_call(
        _edge_net_kernel,
        out_shape=jax.ShapeDtypeStruct((1, D * D), jnp.float32),
    )(e0, We1, be1.reshape(1, EH), We2, be2.reshape(1, D * D))
    return out.reshape(D, D)


def _node_proj_kernel(x_ref, Wp_ref, bp_ref, At_ref, h_ref, proj_ref):
    h = jnp.maximum(
        jnp.dot(x_ref[...], Wp_ref[...], preferred_element_type=jnp.float32)
        + bp_ref[...], 0.0)
    h_ref[...] = h
    proj = jnp.dot(h, At_ref[...], preferred_element_type=jnp.float32)
    proj_ref[...] = jnp.concatenate(
        [proj, jnp.zeros((proj.shape[0], DP - D), jnp.float32)], axis=1)


def _node_proj(x, W_proj, b_proj, A_t):
    blk = 2000
    grid = (N // blk,)
    return pl.pallas_call(
        _node_proj_kernel,
        grid=grid,
        in_specs=[
            pl.BlockSpec((blk, D), lambda i: (i, 0)),
            pl.BlockSpec((D, D), lambda i: (0, 0)),
            pl.BlockSpec((1, D), lambda i: (0, 0)),
            pl.BlockSpec((D, D), lambda i: (0, 0)),
        ],
        out_specs=[
            pl.BlockSpec((blk, D), lambda i: (i, 0)),
            pl.BlockSpec((blk, DP), lambda i: (i, 0)),
        ],
        out_shape=[
            jax.ShapeDtypeStruct((N, D), jnp.float32),
            jax.ShapeDtypeStruct((N, DP), jnp.float32),
        ],
    )(x, W_proj, b_proj.reshape(1, D), A_t)


NBUF = 2


def _sc_body(proj_hbm, src_hbm, dst_hbm, zeros_hbm, out_hbm,
             sg0, sg1, dg0, dg1, r0, r1, agg_sh,
             is0, is1, id0, id1, g0, g1):
    c = lax.axis_index("c")
    s = lax.axis_index("s")
    w = c * NS + s
    rows = [r0, r1]
    sgb = [sg0, sg1]
    dgb = [dg0, dg1]
    isem = [is0, is1]
    jsem = [id0, id1]
    gsem = [g0, g1]

    pltpu.sync_copy(zeros_hbm.at[pl.ds(s * ROWS_PER_TILE, ROWS_PER_TILE)],
                    agg_sh.at[pl.ds(s * ROWS_PER_TILE, ROWS_PER_TILE)])
    @pl.when(s == 0)
    def _():
        pltpu.sync_copy(zeros_hbm.at[pl.ds(0, AGG_ROWS - N_STRIPED)],
                        agg_sh.at[pl.ds(N_STRIPED, AGG_ROWS - N_STRIPED)])

    def idx_off(g, j):
        return w * CHUNK + (g * IG + j) * NW * CHUNK

    def idx_fetch(g, q):
        for j in range(IG):
            pltpu.async_copy(src_hbm.at[pl.ds(idx_off(g, j), CHUNK)],
                             sgb[q].at[j], isem[q])
            pltpu.async_copy(dst_hbm.at[pl.ds(idx_off(g, j), CHUNK)],
                             dgb[q].at[j], jsem[q])

    def idx_wait(g, q):
        for j in range(IG):
            pltpu.make_async_copy(src_hbm.at[pl.ds(idx_off(g, j), CHUNK)],
                                  sgb[q].at[j], isem[q]).wait()
            pltpu.make_async_copy(dst_hbm.at[pl.ds(idx_off(g, j), CHUNK)],
                                  dgb[q].at[j], jsem[q]).wait()

    def gather_start(t, k):
        q = (k // 4) % 2
        r = k % 4
        g = 2 * t + k // 4
        if r == 0:
            idx_wait(g, q)
        if r == 1:
            @pl.when(g + 1 < NIG)
            def _():
                idx_fetch(g + 1, 1 - q)
        pltpu.async_copy(proj_hbm.at[sgb[q].at[r]], rows[k % 2], gsem[k % 2])

    def gather_wait(k):
        q = (k // 4) % 2
        pltpu.make_async_copy(proj_hbm.at[sgb[q].at[k % 4]], rows[k % 2],
                              gsem[k % 2]).wait()

    idx_fetch(0, 0)
    plsc.subcore_barrier()
    gather_start(0, 0)
    gather_start(0, 1)

    def octet(t, carry):
        for k in range(8):
            gather_wait(k)
            pltpu.sync_copy(rows[k % 2],
                            agg_sh.at[dgb[(k // 4) % 2].at[k % 4]], add=True)
            if k < 6:
                gather_start(t, k + 2)
            else:
                @pl.when(t < TGRP - 1)
                def _():
                    gather_start(t + 1, k - 6)
        return carry

    lax.fori_loop(0, TGRP, octet, 0)
    plsc.subcore_barrier()

    pltpu.sync_copy(agg_sh.at[pl.ds(s * ROWS_PER_TILE, ROWS_PER_TILE)],
                    out_hbm.at[c].at[pl.ds(s * ROWS_PER_TILE, ROWS_PER_TILE)])


def _sc_aggregate(proj_pad, src_pad, dst_pad, zeros_hbm):
    mesh = plsc.VectorSubcoreMesh(core_axis_name="c", subcore_axis_name="s")
    k = pl.kernel(
        _sc_body,
        out_type=jax.ShapeDtypeStruct((NC, N_STRIPED, DP), jnp.float32),
        mesh=mesh,
        scratch_types=(
            [pltpu.VMEM((IG, CHUNK), jnp.int32)] * 4
            + [pltpu.VMEM((CHUNK, DP), jnp.float32)] * NBUF
            + [pltpu.VMEM_SHARED((AGG_ROWS, DP), jnp.float32)]
            + [pltpu.SemaphoreType.DMA] * 6
        ),
        compiler_params=pltpu.CompilerParams(use_tc_tiling_on_sc=False),
    )
    return k(proj_pad, src_pad, dst_pad, zeros_hbm)


def _gru_kernel(p_ref, h_ref,
                Wir_ref, Wiz_ref, Win_ref, bi_ref,
                Whr_ref, Whz_ref, Whn_ref, bh_ref, out_ref):
    m = jnp.maximum(p_ref[0, :, :D] + p_ref[1, :, :D], 0.0)
    h = h_ref[...]
    dot = lambda a, b: jnp.dot(a, b, preferred_element_type=jnp.float32)
    r = jax.nn.sigmoid(dot(m, Wir_ref[...]) + bi_ref[0:1, :]
                       + dot(h, Whr_ref[...]) + bh_ref[0:1, :])
    z = jax.nn.sigmoid(dot(m, Wiz_ref[...]) + bi_ref[1:2, :]
                       + dot(h, Whz_ref[...]) + bh_ref[1:2, :])
    n = jnp.tanh(dot(m, Win_ref[...]) + bi_ref[2:3, :]
                 + r * (dot(h, Whn_ref[...]) + bh_ref[2:3, :]))
    out_ref[...] = (1.0 - z) * n + z * h


def _gru(partials, h, W_ih, b_ih, W_hh, b_hh):
    blk = 2000
    grid = (N // blk,)
    Wir, Wiz, Win = W_ih[:, :D], W_ih[:, D:2 * D], W_ih[:, 2 * D:]
    Whr, Whz, Whn = W_hh[:, :D], W_hh[:, D:2 * D], W_hh[:, 2 * D:]
    bi = b_ih.reshape(3, D)
    bh = b_hh.reshape(3, D)
    full = lambda shape: pl.BlockSpec(shape, lambda i: tuple(0 for _ in shape))
    return pl.pallas_call(
        _gru_kernel,
        grid=grid,
        in_specs=[
            pl.BlockSpec((2, blk, DP), lambda i: (0, i, 0)),
            pl.BlockSpec((blk, D), lambda i: (i, 0)),
            full((D, D)), full((D, D)), full((D, D)), full((3, D)),
            full((D, D)), full((D, D)), full((D, D)), full((3, D)),
        ],
        out_specs=pl.BlockSpec((blk, D), lambda i: (i, 0)),
        out_shape=jax.ShapeDtypeStruct((N, D), jnp.float32),
    )(partials, h, Wir, Wiz, Win, bi, Whr, Whz, Whn, bh)


@jax.jit
def kernel(x, edge_index, edge_attr, W_proj, b_proj, We1, be1, We2, be2,
           W_ih, b_ih, W_hh, b_hh):
    A = _edge_net(edge_attr[:1], We1, be1, We2, be2)
    h, proj_pad = _node_proj(x, W_proj, b_proj, A.T)

    pad = E_PAD - E
    src_pad = jnp.concatenate([edge_index[0], jnp.zeros((pad,), jnp.int32)])
    dst_pad = jnp.concatenate(
        [edge_index[1], jnp.full((pad,), JUNK_ROW, jnp.int32)])
    zeros_hbm = jnp.zeros((N_STRIPED, DP), jnp.float32)

    partials = _sc_aggregate(proj_pad, src_pad, dst_pad, zeros_hbm)
    return _gru(partials, h, W_ih, b_ih, W_hh, b_hh)
